# baseline (speedup 1.0000x reference)
import functools

import jax
import jax.numpy as jnp
from jax import lax
from jax.experimental import pallas as pl
from jax.experimental.pallas import tpu as pltpu
from jax.experimental.pallas import tpu_sc as plsc

NUM_TOKENS = 16384
VOCAB = 1_000_000
EMBED = 64
OUT = 128
LANES = 16
NC, NS = 2, 16
NW = NC * NS
TOK_PER_W = NUM_TOKENS // NW
HSIZE = 1 << 20
SLICE_PER_S = HSIZE // NS
ZB = 16384

_mesh = plsc.VectorSubcoreMesh(core_axis_name="c", subcore_axis_name="s")


@functools.partial(
    pl.kernel,
    out_type=jax.ShapeDtypeStruct((NC, HSIZE), jnp.float32),
    mesh=_mesh,
    scratch_types=[
        pltpu.VMEM((TOK_PER_W,), jnp.int32),
        pltpu.VMEM((TOK_PER_W,), jnp.float32),
        pltpu.VMEM((ZB,), jnp.float32),
        pltpu.VMEM_SHARED((HSIZE,), jnp.float32),
    ],
)
def _histogram(idx_hbm, out_hbm, idx_v, ones_v, zblk_v, hist_sh):
    cid = lax.axis_index("c")
    sid = lax.axis_index("s")

    zv = jnp.zeros((LANES,), jnp.float32)
    ov = jnp.full((LANES,), 1.0, jnp.float32)

    for i in range(ZB // LANES):
        zblk_v[pl.ds(i * LANES, LANES)] = zv
    for i in range(TOK_PER_W // LANES):
        ones_v[pl.ds(i * LANES, LANES)] = ov

    base = sid * SLICE_PER_S
    for r in range(SLICE_PER_S // ZB):
        pltpu.sync_copy(zblk_v, hist_sh.at[pl.ds(base + r * ZB, ZB)])
    plsc.subcore_barrier()

    pltpu.sync_copy(idx_hbm.at[cid, sid], idx_v)
    pltpu.sync_copy(ones_v, hist_sh.at[idx_v], add=True)
    plsc.subcore_barrier()

    for r in range(SLICE_PER_S // ZB):
        sl = pl.ds(base + r * ZB, ZB)
        pltpu.sync_copy(hist_sh.at[sl], out_hbm.at[cid, sl])


BK = 32768
NSC = 262144
CW = 512
V_PER_W = NSC // NW
CH_PER_W = V_PER_W // CW
SC_BLOCKS = NSC // BK
NSTEP = (VOCAB - NSC + BK - 1) // BK


@functools.partial(
    pl.kernel,
    out_type=jax.ShapeDtypeStruct((NW, EMBED, LANES), jnp.float32),
    mesh=_mesh,
    scratch_types=[
        pltpu.VMEM((2, EMBED, CW), jnp.float32),
        pltpu.VMEM((2, NC, CW), jnp.float32),
        pltpu.VMEM((EMBED, LANES), jnp.float32),
        pltpu.SemaphoreType.DMA,
    ],
)
def _scan_sc(counts_hbm, tabT_hbm, out_hbm, tab_v, cnt_v, acc_v, sem):
    wid = lax.axis_index("s") * NC + lax.axis_index("c")
    v0 = wid * V_PER_W

    zv = jnp.zeros((LANES,), jnp.float32)
    for e in range(EMBED):
        acc_v[e, :] = zv

    def start(i, b):
        pltpu.async_copy(tabT_hbm.at[:, pl.ds(v0 + i * CW, CW)], tab_v.at[b], sem)
        pltpu.async_copy(counts_hbm.at[:, pl.ds(v0 + i * CW, CW)], cnt_v.at[b], sem)

    start(0, 0)
    start(1, 1)

    def body2(j, _):
        for b in range(2):
            i = 2 * j + b
            pltpu.make_async_copy(tabT_hbm.at[:, pl.ds(0, CW)], tab_v.at[b], sem).wait()
            pltpu.make_async_copy(counts_hbm.at[:, pl.ds(0, CW)], cnt_v.at[b], sem).wait()
            for eg in range(EMBED // 8):

                def gstep(g, ps):
                    base = g * LANES
                    cs = cnt_v[b, 0, pl.ds(base, LANES)] + cnt_v[b, 1, pl.ds(base, LANES)]
                    return tuple(
                        ps[r] + tab_v[b, eg * 8 + r, pl.ds(base, LANES)] * cs
                        for r in range(8)
                    )

                ps = lax.fori_loop(0, CW // LANES, gstep, (zv,) * 8)
                for r in range(8):
                    e = eg * 8 + r
                    acc_v[e, :] = acc_v[e, :] + ps[r]

            @pl.when(i + 2 < CH_PER_W)
            def _():
                pltpu.async_copy(
                    tabT_hbm.at[:, pl.ds(v0 + (i + 2) * CW, CW)], tab_v.at[b], sem
                )
                pltpu.async_copy(
                    counts_hbm.at[:, pl.ds(v0 + (i + 2) * CW, CW)], cnt_v.at[b], sem
                )
        return 0

    lax.fori_loop(0, CH_PER_W // 2, body2, 0)
    pltpu.sync_copy(acc_v, out_hbm.at[wid])


def _scan_tc(tabT_ref, c_ref, o_ref, acc_ref):
    k = pl.program_id(0)

    @pl.when(k == 0)
    def _():
        acc_ref[...] = jnp.zeros_like(acc_ref)

    cb = (c_ref[0, :] + c_ref[1, :]).reshape(1, BK)
    acc_ref[...] += tabT_ref[...] * cb

    @pl.when(k == NSTEP - 1)
    def _():
        o_ref[...] = jnp.sum(acc_ref[...], axis=1).reshape(1, EMBED)


def _head(tc_ref, sc_ref, w_ref, b_ref, o_ref):
    ps = jnp.sum(sc_ref[...].reshape(NW, EMBED, LANES), axis=(0, 2)).reshape(1, EMBED)
    s = (tc_ref[...] + ps) * (1.0 / NUM_TOKENS)
    o_ref[...] = (
        lax.dot_general(
            s, w_ref[...], (((1,), (1,)), ((), ())),
            preferred_element_type=jnp.float32,
        )
        + b_ref[...]
    )


def kernel(x, emb_table, fc_weight, fc_bias, extra_bias):
    idx = x.reshape(NC, NS, TOK_PER_W)
    counts = _histogram(idx)

    tabT = emb_table.T
    partial_sc = _scan_sc(counts, tabT)

    acc_tc = pl.pallas_call(
        _scan_tc,
        grid=(NSTEP,),
        in_specs=[
            pl.BlockSpec((EMBED, BK), lambda k: (0, k + SC_BLOCKS)),
            pl.BlockSpec((NC, BK), lambda k: (0, k + SC_BLOCKS)),
        ],
        out_specs=pl.BlockSpec((1, EMBED), lambda k: (0, 0)),
        out_shape=jax.ShapeDtypeStruct((1, EMBED), jnp.float32),
        scratch_shapes=[pltpu.VMEM((EMBED, BK), jnp.float32)],
    )(tabT, counts)

    bias = (fc_bias + extra_bias).reshape(1, OUT)
    out = pl.pallas_call(
        _head,
        in_specs=[
            pl.BlockSpec((1, EMBED), lambda: (0, 0)),
            pl.BlockSpec((NW * EMBED, LANES), lambda: (0, 0)),
            pl.BlockSpec((OUT, EMBED), lambda: (0, 0)),
            pl.BlockSpec((1, OUT), lambda: (0, 0)),
        ],
        out_specs=pl.BlockSpec((1, OUT), lambda: (0, 0)),
        out_shape=jax.ShapeDtypeStruct((1, OUT), jnp.float32),
    )(acc_tc, partial_sc.reshape(NW * EMBED, LANES), fc_weight, bias)
    return out


# speedup vs baseline: 1.0323x; 1.0323x over previous
import functools

import jax
import jax.numpy as jnp
from jax import lax
from jax.experimental import pallas as pl
from jax.experimental.pallas import tpu as pltpu
from jax.experimental.pallas import tpu_sc as plsc

NUM_TOKENS = 16384
VOCAB = 1_000_000
EMBED = 64
OUT = 128
LANES = 16
NC, NS = 2, 16
NW = NC * NS
TOK_PER_W = NUM_TOKENS // NW
HSIZE = 1 << 20
SLICE_PER_S = HSIZE // NS
ZB = 16384

_mesh = plsc.VectorSubcoreMesh(core_axis_name="c", subcore_axis_name="s")


@functools.partial(
    pl.kernel,
    out_type=jax.ShapeDtypeStruct((NC, HSIZE), jnp.float32),
    mesh=_mesh,
    scratch_types=[
        pltpu.VMEM((TOK_PER_W,), jnp.int32),
        pltpu.VMEM((TOK_PER_W,), jnp.float32),
        pltpu.VMEM((ZB,), jnp.float32),
        pltpu.VMEM_SHARED((HSIZE,), jnp.float32),
        pltpu.SemaphoreType.DMA,
        pltpu.SemaphoreType.DMA,
    ],
)
def _histogram(idx_hbm, out_hbm, idx_v, ones_v, zblk_v, hist_sh, semz, semi):
    cid = lax.axis_index("c")
    sid = lax.axis_index("s")
    wid = cid * NS + sid

    zv = jnp.zeros((LANES,), jnp.float32)
    ov = jnp.full((LANES,), 1.0, jnp.float32)

    for i in range(ZB // LANES):
        zblk_v[pl.ds(i * LANES, LANES)] = zv

    base = sid * SLICE_PER_S
    zcs = [
        pltpu.async_copy(zblk_v, hist_sh.at[pl.ds(base + r * ZB, ZB)], semz)
        for r in range(SLICE_PER_S // ZB)
    ]
    ic = pltpu.async_copy(idx_hbm.at[pl.ds(wid * TOK_PER_W, TOK_PER_W)], idx_v, semi)

    for i in range(TOK_PER_W // LANES):
        ones_v[pl.ds(i * LANES, LANES)] = ov

    for c in zcs:
        c.wait()
    ic.wait()
    plsc.subcore_barrier()

    pltpu.sync_copy(ones_v, hist_sh.at[idx_v], add=True)
    plsc.subcore_barrier()

    ocs = [
        pltpu.async_copy(
            hist_sh.at[pl.ds(base + r * ZB, ZB)],
            out_hbm.at[cid, pl.ds(base + r * ZB, ZB)],
            semz,
        )
        for r in range(SLICE_PER_S // ZB)
    ]
    for c in ocs:
        c.wait()


BK = 65536
NSTEP = (VOCAB + BK - 1) // BK


def _matvec_head(tabT_ref, c_ref, w_ref, b_ref, o_ref, acc_ref):
    k = pl.program_id(0)

    @pl.when(k == 0)
    def _():
        acc_ref[...] = jnp.zeros_like(acc_ref)

    cb = (c_ref[0, :] + c_ref[1, :]).reshape(1, BK)
    acc_ref[...] += tabT_ref[...] * cb

    @pl.when(k == NSTEP - 1)
    def _():
        s = jnp.sum(acc_ref[...], axis=1).reshape(1, EMBED) * (1.0 / NUM_TOKENS)
        o_ref[...] = (
            lax.dot_general(
                s, w_ref[...], (((1,), (1,)), ((), ())),
                preferred_element_type=jnp.float32,
            )
            + b_ref[...]
        )


def kernel(x, emb_table, fc_weight, fc_bias, extra_bias):
    counts = _histogram(x)

    tabT = emb_table.T
    bias = (fc_bias + extra_bias).reshape(1, OUT)
    out = pl.pallas_call(
        _matvec_head,
        grid=(NSTEP,),
        in_specs=[
            pl.BlockSpec((EMBED, BK), lambda k: (0, k)),
            pl.BlockSpec((NC, BK), lambda k: (0, k)),
            pl.BlockSpec((OUT, EMBED), lambda k: (0, 0)),
            pl.BlockSpec((1, OUT), lambda k: (0, 0)),
        ],
        out_specs=pl.BlockSpec((1, OUT), lambda k: (0, 0)),
        out_shape=jax.ShapeDtypeStruct((1, OUT), jnp.float32),
        scratch_shapes=[pltpu.VMEM((EMBED, BK), jnp.float32)],
    )(tabT, counts, fc_weight, bias)
    return out
